# baseline (speedup 1.0000x reference)
import jax
import jax.numpy as jnp
from jax import lax
from jax.experimental import pallas as pl
from jax.experimental.pallas import tpu as pltpu
from jax.experimental.pallas import tpu_sc as plsc

N_IN = 2048
N_OUT = 2048
HALF_ROWS = 32
HALF_W = HALF_ROWS * N_IN
SUB = 8192
STRIDE = 2048


def _sc_densify_body(nnz, coarse_hbm, ind_hbm, vals_hbm,
                     trow_hbm, tcol_hbm, tval_hbm, z_hbm, w_hbm,
                     bnd_v, win_v, rcA, vA, rcB, vB,
                     tr_v, tc_v, tv_v, wbuf, semA, semB, semZ):
    c = lax.axis_index("c")
    s = lax.axis_index("s")
    t2 = (s * 2 + c) * 2
    pltpu.async_copy(z_hbm, wbuf, semZ)
    pltpu.sync_copy(coarse_hbm, bnd_v)
    pltpu.sync_copy(trow_hbm, tr_v)
    pltpu.sync_copy(tcol_hbm, tc_v)
    pltpu.sync_copy(tval_hbm, tv_v)
    zero16 = jnp.zeros((16,), jnp.float32)
    lanes0 = lax.iota(jnp.int32, 16)
    last128 = (nnz - SUB) & ~127
    wsmax = (nnz - STRIDE) & ~127
    TAIL = 128

    jcv = bnd_v[pl.ds(t2, 16)]
    ws_list, handles = [], []
    for k in range(3):
        ws = jnp.minimum(jnp.maximum((jcv[k] - 1) * STRIDE, 0), wsmax)
        ws = pl.multiple_of(ws, 128)
        ws_list.append(ws)
        handles.append(pltpu.async_copy(
            ind_hbm.at[:, pl.ds(ws, STRIDE)],
            win_v.at[:, pl.ds(k * STRIDE, STRIDE)], semA))
    for hd in handles:
        hd.wait()
    bnds = []
    for k in range(3):
        q = (t2 + k) * HALF_ROWS
        def cbody(i, acc, k=k, q=q):
            v = win_v[0, pl.ds(k * STRIDE + i * 16, 16)]
            return acc + plsc.all_reduce_population_count(v < q)
        acc = lax.fori_loop(0, STRIDE // 16, cbody,
                            jnp.zeros((16,), jnp.int32), unroll=8)
        for i in range(TAIL // 16):
            tpos = lanes0 + (nnz - TAIL) + i * 16
            acc = acc + plsc.all_reduce_population_count(
                (tpos >= ws_list[k] + STRIDE)
                & (tr_v[pl.ds(i * 16, 16)] < q))
        bnds.append(ws_list[k] + acc[0])

    def starts(j, s0):
        rstart = jnp.minimum(s0 + j * SUB, last128)
        return pl.multiple_of(rstart, 128)

    def issue(rstart, rc_b, v_b, sem):
        pltpu.async_copy(ind_hbm.at[:, pl.ds(rstart, SUB)], rc_b, sem)
        pltpu.async_copy(vals_hbm.at[pl.ds(rstart, SUB)], v_b, sem)

    def drain(rc_b, v_b, sem):
        pltpu.make_async_copy(ind_hbm.at[:, pl.ds(0, SUB)], rc_b, sem).wait()
        pltpu.make_async_copy(vals_hbm.at[pl.ds(0, SUB)], v_b, sem).wait()

    for h in range(2):
        blk = t2 + h
        b0, b1 = bnds[h], bnds[h + 1]
        row0 = blk * HALF_ROWS
        s0 = b0 & ~127

        issue(starts(0, s0), rcA, vA, semA)
        pltpu.make_async_copy(z_hbm, wbuf, semZ).wait()

        def scatter(j, rc_b, v_b):
            rstart = starts(j, s0)
            lo = b0 - rstart
            hi = b1 - rstart

            def scat(i, _):
                lane = lanes0 + i * 16
                m = (lane >= lo) & (lane < hi)
                ri = rc_b[0, pl.ds(i * 16, 16)] - row0
                ci = rc_b[1, pl.ds(i * 16, 16)]
                v = v_b[pl.ds(i * 16, 16)]
                plsc.store_scatter(wbuf, [ri, ci], v, mask=m)
                return 0
            lax.fori_loop(0, SUB // 16, scat, 0, unroll=16)

        nsub = (b1 - s0 + SUB - 1) // SUB

        def pair(jj, _):
            j0 = 2 * jj
            issue(starts(j0 + 1, s0), rcB, vB, semB)
            drain(rcA, vA, semA)
            scatter(j0, rcA, vA)
            issue(starts(j0 + 2, s0), rcA, vA, semA)
            drain(rcB, vB, semB)
            scatter(j0 + 1, rcB, vB)
            return 0
        pairs = nsub // 2
        lax.fori_loop(0, pairs, pair, 0)
        drain(rcA, vA, semA)
        lax.cond(nsub % 2 == 1,
                 lambda: scatter(2 * pairs, rcA, vA), lambda: None)

        for i in range(TAIL // 16):
            trow = tr_v[pl.ds(i * 16, 16)]
            tmask = (trow >= row0) & (trow < row0 + HALF_ROWS)
            plsc.store_scatter(
                wbuf, [trow - row0, tc_v[pl.ds(i * 16, 16)]],
                tv_v[pl.ds(i * 16, 16)], mask=tmask)

        pltpu.sync_copy(wbuf, w_hbm.at[pl.ds(row0, HALF_ROWS), :])
        if h == 0:
            pltpu.async_copy(z_hbm, wbuf, semZ)


def _densify(coarse, ind, vals, trow, tcol, tval):
    mesh = plsc.VectorSubcoreMesh(core_axis_name="c", subcore_axis_name="s")
    nnz = vals.shape[0]
    body = lambda *refs: _sc_densify_body(nnz, *refs)
    return pl.kernel(
        body,
        out_type=jax.ShapeDtypeStruct((N_OUT, N_IN), jnp.float32),
        mesh=mesh,
        scratch_types=[
            pltpu.VMEM((128,), jnp.int32),
            pltpu.VMEM((2, 3 * STRIDE), jnp.int32),
            pltpu.VMEM((2, SUB), jnp.int32),
            pltpu.VMEM((SUB,), jnp.float32),
            pltpu.VMEM((2, SUB), jnp.int32),
            pltpu.VMEM((SUB,), jnp.float32),
            pltpu.VMEM((128,), jnp.int32),
            pltpu.VMEM((128,), jnp.int32),
            pltpu.VMEM((128,), jnp.float32),
            pltpu.VMEM((HALF_ROWS, N_IN), jnp.float32),
            pltpu.SemaphoreType.DMA,
            pltpu.SemaphoreType.DMA,
            pltpu.SemaphoreType.DMA,
        ],
        compiler_params=pltpu.CompilerParams(needs_layout_passes=False),
    )(coarse, ind, vals, trow, tcol, tval,
      jnp.zeros((HALF_ROWS, N_IN), jnp.float32))


BM = 512


def _mm_body(x_ref, w_ref, b_ref, o_ref, wbf_ref):
    @pl.when(pl.program_id(0) == 0)
    def _():
        wbf_ref[...] = w_ref[...].astype(jnp.bfloat16)
    xb = x_ref[...].astype(jnp.bfloat16)
    acc = lax.dot_general(xb, wbf_ref[...], (((1,), (1,)), ((), ())),
                          preferred_element_type=jnp.float32)
    o_ref[...] = acc + b_ref[...]


def _matmul(x, w, bias2):
    batch = x.shape[0]
    return pl.pallas_call(
        _mm_body,
        grid=(batch // BM,),
        in_specs=[
            pl.BlockSpec((BM, N_IN), lambda i: (i, 0)),
            pl.BlockSpec((N_OUT, N_IN), lambda i: (0, 0)),
            pl.BlockSpec((1, N_OUT), lambda i: (0, 0)),
        ],
        out_specs=pl.BlockSpec((BM, N_OUT), lambda i: (i, 0)),
        out_shape=jax.ShapeDtypeStruct((batch, N_OUT), jnp.float32),
        scratch_shapes=[pltpu.VMEM((N_OUT, N_IN), jnp.bfloat16)],
    )(x, w, bias2)


def kernel(in_values, values, indices, bias):
    ind = indices.astype(jnp.int32)
    nnz = values.shape[0]
    q = jnp.arange(0, N_OUT + 1, HALF_ROWS, dtype=jnp.int32)
    r_sub = indices[0, ::STRIDE].astype(jnp.int32)
    jc = jnp.sum(r_sub[None, :] < q[:, None], axis=1).astype(jnp.int32)
    coarse = jnp.pad(jc, (0, 128 - jc.shape[0]))
    trow = ind[0, nnz - 128:]
    tcol = ind[1, nnz - 128:]
    tval = values[nnz - 128:]
    w = _densify(coarse, ind, values, trow, tcol, tval)
    return _matmul(in_values, w, bias.reshape(1, N_OUT))


# speedup vs baseline: 1.2808x; 1.2808x over previous
import jax
import jax.numpy as jnp
from jax import lax
from jax.experimental import pallas as pl
from jax.experimental.pallas import tpu as pltpu
from jax.experimental.pallas import tpu_sc as plsc

N_IN = 2048
N_OUT = 2048
HALF_ROWS = 32
HALF_W = HALF_ROWS * N_IN
SUB = 8192
STRIDE = 2048


def _sc_densify_body(nnz, coarse_hbm, ind_hbm, vals_hbm,
                     trow_hbm, tcol_hbm, tval_hbm, z_hbm, w_hbm,
                     bnd_v, win_v, rcA, vA, rcB, vB,
                     tr_v, tc_v, tv_v, wbuf, semA, semB, semZ):
    c = lax.axis_index("c")
    s = lax.axis_index("s")
    t2 = (s * 2 + c) * 2
    pltpu.async_copy(z_hbm, wbuf, semZ)
    pltpu.sync_copy(coarse_hbm, bnd_v)
    pltpu.sync_copy(trow_hbm, tr_v)
    pltpu.sync_copy(tcol_hbm, tc_v)
    pltpu.sync_copy(tval_hbm, tv_v)
    zero16 = jnp.zeros((16,), jnp.float32)
    lanes0 = lax.iota(jnp.int32, 16)
    last128 = (nnz - SUB) & ~127
    wsmax = (nnz - STRIDE) & ~127
    TAIL = 128

    jcv = bnd_v[pl.ds(t2, 16)]
    ws_list, handles = [], []
    for k in range(3):
        ws = jnp.minimum(jnp.maximum((jcv[k] - 1) * STRIDE, 0), wsmax)
        ws = pl.multiple_of(ws, 128)
        ws_list.append(ws)
        handles.append(pltpu.async_copy(
            ind_hbm.at[:, pl.ds(ws, STRIDE)],
            win_v.at[:, pl.ds(k * STRIDE, STRIDE)], semA))
    for hd in handles:
        hd.wait()
    bnds = []
    for k in range(3):
        q = (t2 + k) * HALF_ROWS
        def cbody(i, acc, k=k, q=q):
            v = win_v[0, pl.ds(k * STRIDE + i * 16, 16)]
            return acc + plsc.all_reduce_population_count(v < q)
        acc = lax.fori_loop(0, STRIDE // 16, cbody,
                            jnp.zeros((16,), jnp.int32), unroll=8)
        for i in range(TAIL // 16):
            tpos = lanes0 + (nnz - TAIL) + i * 16
            acc = acc + plsc.all_reduce_population_count(
                (tpos >= ws_list[k] + STRIDE)
                & (tr_v[pl.ds(i * 16, 16)] < q))
        bnds.append(ws_list[k] + acc[0])

    def starts(j, s0):
        rstart = jnp.minimum(s0 + j * SUB, last128)
        return pl.multiple_of(rstart, 128)

    def issue(rstart, rc_b, v_b, sem):
        pltpu.async_copy(ind_hbm.at[:, pl.ds(rstart, SUB)], rc_b, sem)
        pltpu.async_copy(vals_hbm.at[pl.ds(rstart, SUB)], v_b, sem)

    def drain(rc_b, v_b, sem):
        pltpu.make_async_copy(ind_hbm.at[:, pl.ds(0, SUB)], rc_b, sem).wait()
        pltpu.make_async_copy(vals_hbm.at[pl.ds(0, SUB)], v_b, sem).wait()

    for h in range(2):
        blk = t2 + h
        b0, b1 = bnds[h], bnds[h + 1]
        row0 = blk * HALF_ROWS
        s0 = b0 & ~127

        issue(starts(0, s0), rcA, vA, semA)
        pltpu.make_async_copy(z_hbm, wbuf, semZ).wait()

        def scatter(j, rc_b, v_b):
            rstart = starts(j, s0)
            lo = b0 - rstart
            hi = b1 - rstart

            G = 8

            def scat(i, _):
                b = i * (G * 16)
                rs = [rc_b[0, pl.ds(b + g * 16, 16)] for g in range(G)]
                cs = [rc_b[1, pl.ds(b + g * 16, 16)] for g in range(G)]
                vs = [v_b[pl.ds(b + g * 16, 16)] for g in range(G)]
                for g in range(G):
                    lane = lanes0 + (b + g * 16)
                    m = (lane >= lo) & (lane < hi)
                    plsc.store_scatter(wbuf, [rs[g] - row0, cs[g]], vs[g],
                                       mask=m)
                return 0
            lax.fori_loop(0, SUB // (G * 16), scat, 0, unroll=2)

        nsub = (b1 - s0 + SUB - 1) // SUB

        def pair(jj, _):
            j0 = 2 * jj
            issue(starts(j0 + 1, s0), rcB, vB, semB)
            drain(rcA, vA, semA)
            scatter(j0, rcA, vA)
            issue(starts(j0 + 2, s0), rcA, vA, semA)
            drain(rcB, vB, semB)
            scatter(j0 + 1, rcB, vB)
            return 0
        pairs = nsub // 2
        lax.fori_loop(0, pairs, pair, 0)
        drain(rcA, vA, semA)
        lax.cond(nsub % 2 == 1,
                 lambda: scatter(2 * pairs, rcA, vA), lambda: None)

        for i in range(TAIL // 16):
            trow = tr_v[pl.ds(i * 16, 16)]
            tmask = (trow >= row0) & (trow < row0 + HALF_ROWS)
            plsc.store_scatter(
                wbuf, [trow - row0, tc_v[pl.ds(i * 16, 16)]],
                tv_v[pl.ds(i * 16, 16)], mask=tmask)

        pltpu.sync_copy(wbuf, w_hbm.at[pl.ds(row0, HALF_ROWS), :])
        if h == 0:
            pltpu.async_copy(z_hbm, wbuf, semZ)


def _densify(coarse, ind, vals, trow, tcol, tval):
    mesh = plsc.VectorSubcoreMesh(core_axis_name="c", subcore_axis_name="s")
    nnz = vals.shape[0]
    body = lambda *refs: _sc_densify_body(nnz, *refs)
    return pl.kernel(
        body,
        out_type=jax.ShapeDtypeStruct((N_OUT, N_IN), jnp.float32),
        mesh=mesh,
        scratch_types=[
            pltpu.VMEM((128,), jnp.int32),
            pltpu.VMEM((2, 3 * STRIDE), jnp.int32),
            pltpu.VMEM((2, SUB), jnp.int32),
            pltpu.VMEM((SUB,), jnp.float32),
            pltpu.VMEM((2, SUB), jnp.int32),
            pltpu.VMEM((SUB,), jnp.float32),
            pltpu.VMEM((128,), jnp.int32),
            pltpu.VMEM((128,), jnp.int32),
            pltpu.VMEM((128,), jnp.float32),
            pltpu.VMEM((HALF_ROWS, N_IN), jnp.float32),
            pltpu.SemaphoreType.DMA,
            pltpu.SemaphoreType.DMA,
            pltpu.SemaphoreType.DMA,
        ],
        compiler_params=pltpu.CompilerParams(needs_layout_passes=False),
    )(coarse, ind, vals, trow, tcol, tval,
      jnp.zeros((HALF_ROWS, N_IN), jnp.float32))


BM = 512


def _mm_body(x_ref, w_ref, b_ref, o_ref, wbf_ref):
    @pl.when(pl.program_id(0) == 0)
    def _():
        wbf_ref[...] = w_ref[...].astype(jnp.bfloat16)
    xb = x_ref[...].astype(jnp.bfloat16)
    acc = lax.dot_general(xb, wbf_ref[...], (((1,), (1,)), ((), ())),
                          preferred_element_type=jnp.float32)
    o_ref[...] = acc + b_ref[...]


def _matmul(x, w, bias2):
    batch = x.shape[0]
    return pl.pallas_call(
        _mm_body,
        grid=(batch // BM,),
        in_specs=[
            pl.BlockSpec((BM, N_IN), lambda i: (i, 0)),
            pl.BlockSpec((N_OUT, N_IN), lambda i: (0, 0)),
            pl.BlockSpec((1, N_OUT), lambda i: (0, 0)),
        ],
        out_specs=pl.BlockSpec((BM, N_OUT), lambda i: (i, 0)),
        out_shape=jax.ShapeDtypeStruct((batch, N_OUT), jnp.float32),
        scratch_shapes=[pltpu.VMEM((N_OUT, N_IN), jnp.bfloat16)],
    )(x, w, bias2)


def kernel(in_values, values, indices, bias):
    ind = indices.astype(jnp.int32)
    nnz = values.shape[0]
    q = jnp.arange(0, N_OUT + 1, HALF_ROWS, dtype=jnp.int32)
    r_sub = indices[0, ::STRIDE].astype(jnp.int32)
    jc = jnp.sum(r_sub[None, :] < q[:, None], axis=1).astype(jnp.int32)
    coarse = jnp.pad(jc, (0, 128 - jc.shape[0]))
    trow = ind[0, nnz - 128:]
    tcol = ind[1, nnz - 128:]
    tval = values[nnz - 128:]
    w = _densify(coarse, ind, values, trow, tcol, tval)
    return _matmul(in_values, w, bias.reshape(1, N_OUT))


# speedup vs baseline: 1.2951x; 1.0112x over previous
import jax
import jax.numpy as jnp
from jax import lax
from jax.experimental import pallas as pl
from jax.experimental.pallas import tpu as pltpu
from jax.experimental.pallas import tpu_sc as plsc

N_IN = 2048
N_OUT = 2048
HALF_ROWS = 32
HALF_W = HALF_ROWS * N_IN
SUB = 8192
STRIDE = 2048


def _sc_densify_body(nnz, coarse_hbm, ind_hbm, vals_hbm,
                     trow_hbm, tcol_hbm, tval_hbm, z_hbm, w_hbm,
                     bnd_v, win_v, rcA, vA, rcB, vB,
                     tr_v, tc_v, tv_v, wbuf, semA, semB, semZ):
    c = lax.axis_index("c")
    s = lax.axis_index("s")
    t2 = (s * 2 + c) * 2
    pltpu.async_copy(z_hbm, wbuf, semZ)
    pltpu.sync_copy(coarse_hbm, bnd_v)
    pltpu.sync_copy(trow_hbm, tr_v)
    pltpu.sync_copy(tcol_hbm, tc_v)
    pltpu.sync_copy(tval_hbm, tv_v)
    zero16 = jnp.zeros((16,), jnp.float32)
    lanes0 = lax.iota(jnp.int32, 16)
    last128 = (nnz - SUB) & ~127
    wsmax = (nnz - STRIDE) & ~127
    TAIL = 128

    jcv = bnd_v[pl.ds(t2, 16)]
    ws_list, handles = [], []
    for k in range(3):
        ws = jnp.minimum(jnp.maximum((jcv[k] - 1) * STRIDE, 0), wsmax)
        ws = pl.multiple_of(ws, 128)
        ws_list.append(ws)
        handles.append(pltpu.async_copy(
            ind_hbm.at[:, pl.ds(ws, STRIDE)],
            win_v.at[:, pl.ds(k * STRIDE, STRIDE)], semA))
    for hd in handles:
        hd.wait()
    bnds = []
    for k in range(3):
        q = (t2 + k) * HALF_ROWS
        def cbody(i, acc, k=k, q=q):
            v = win_v[0, pl.ds(k * STRIDE + i * 16, 16)]
            return acc + plsc.all_reduce_population_count(v < q)
        acc = lax.fori_loop(0, STRIDE // 16, cbody,
                            jnp.zeros((16,), jnp.int32), unroll=8)
        for i in range(TAIL // 16):
            tpos = lanes0 + (nnz - TAIL) + i * 16
            acc = acc + plsc.all_reduce_population_count(
                (tpos >= ws_list[k] + STRIDE)
                & (tr_v[pl.ds(i * 16, 16)] < q))
        bnds.append(ws_list[k] + acc[0])

    def starts(j, s0):
        rstart = jnp.minimum(s0 + j * SUB, last128)
        return pl.multiple_of(rstart, 128)

    def issue(rstart, rc_b, v_b, sem):
        pltpu.async_copy(ind_hbm.at[:, pl.ds(rstart, SUB)], rc_b, sem)
        pltpu.async_copy(vals_hbm.at[pl.ds(rstart, SUB)], v_b, sem)

    def drain(rc_b, v_b, sem):
        pltpu.make_async_copy(ind_hbm.at[:, pl.ds(0, SUB)], rc_b, sem).wait()
        pltpu.make_async_copy(vals_hbm.at[pl.ds(0, SUB)], v_b, sem).wait()

    for h in range(2):
        blk = t2 + h
        b0, b1 = bnds[h], bnds[h + 1]
        row0 = blk * HALF_ROWS
        s0 = b0 & ~127

        issue(starts(0, s0), rcA, vA, semA)
        pltpu.make_async_copy(z_hbm, wbuf, semZ).wait()

        def scatter(j, rc_b, v_b):
            rstart = starts(j, s0)
            lo = b0 - rstart
            hi = b1 - rstart

            G = 8

            def scat_masked(i, _):
                b = i * (G * 16)
                rs = [rc_b[0, pl.ds(b + g * 16, 16)] for g in range(G)]
                cs = [rc_b[1, pl.ds(b + g * 16, 16)] for g in range(G)]
                vs = [v_b[pl.ds(b + g * 16, 16)] for g in range(G)]
                for g in range(G):
                    lane = lanes0 + (b + g * 16)
                    m = (lane >= lo) & (lane < hi)
                    plsc.store_scatter(wbuf, [rs[g] - row0, cs[g]], vs[g],
                                       mask=m)
                return 0

            def scat_full(i, _):
                b = i * (G * 16)
                rs = [rc_b[0, pl.ds(b + g * 16, 16)] for g in range(G)]
                cs = [rc_b[1, pl.ds(b + g * 16, 16)] for g in range(G)]
                vs = [v_b[pl.ds(b + g * 16, 16)] for g in range(G)]
                for g in range(G):
                    plsc.store_scatter(wbuf, [rs[g] - row0, cs[g]], vs[g])
                return 0

            lax.cond(
                (lo <= 0) & (hi >= SUB),
                lambda: lax.fori_loop(0, SUB // (G * 16), scat_full, 0,
                                      unroll=2),
                lambda: lax.fori_loop(0, SUB // (G * 16), scat_masked, 0,
                                      unroll=2))

        nsub = (b1 - s0 + SUB - 1) // SUB

        def pair(jj, _):
            j0 = 2 * jj
            issue(starts(j0 + 1, s0), rcB, vB, semB)
            drain(rcA, vA, semA)
            scatter(j0, rcA, vA)
            issue(starts(j0 + 2, s0), rcA, vA, semA)
            drain(rcB, vB, semB)
            scatter(j0 + 1, rcB, vB)
            return 0
        pairs = nsub // 2
        lax.fori_loop(0, pairs, pair, 0)
        drain(rcA, vA, semA)
        lax.cond(nsub % 2 == 1,
                 lambda: scatter(2 * pairs, rcA, vA), lambda: None)

        for i in range(TAIL // 16):
            trow = tr_v[pl.ds(i * 16, 16)]
            tmask = (trow >= row0) & (trow < row0 + HALF_ROWS)
            plsc.store_scatter(
                wbuf, [trow - row0, tc_v[pl.ds(i * 16, 16)]],
                tv_v[pl.ds(i * 16, 16)], mask=tmask)

        pltpu.sync_copy(wbuf, w_hbm.at[pl.ds(row0, HALF_ROWS), :])
        if h == 0:
            pltpu.async_copy(z_hbm, wbuf, semZ)


def _densify(coarse, ind, vals, trow, tcol, tval):
    mesh = plsc.VectorSubcoreMesh(core_axis_name="c", subcore_axis_name="s")
    nnz = vals.shape[0]
    body = lambda *refs: _sc_densify_body(nnz, *refs)
    return pl.kernel(
        body,
        out_type=jax.ShapeDtypeStruct((N_OUT, N_IN), jnp.float32),
        mesh=mesh,
        scratch_types=[
            pltpu.VMEM((128,), jnp.int32),
            pltpu.VMEM((2, 3 * STRIDE), jnp.int32),
            pltpu.VMEM((2, SUB), jnp.int32),
            pltpu.VMEM((SUB,), jnp.float32),
            pltpu.VMEM((2, SUB), jnp.int32),
            pltpu.VMEM((SUB,), jnp.float32),
            pltpu.VMEM((128,), jnp.int32),
            pltpu.VMEM((128,), jnp.int32),
            pltpu.VMEM((128,), jnp.float32),
            pltpu.VMEM((HALF_ROWS, N_IN), jnp.float32),
            pltpu.SemaphoreType.DMA,
            pltpu.SemaphoreType.DMA,
            pltpu.SemaphoreType.DMA,
        ],
        compiler_params=pltpu.CompilerParams(needs_layout_passes=False),
    )(coarse, ind, vals, trow, tcol, tval,
      jnp.zeros((HALF_ROWS, N_IN), jnp.float32))


BM = 512


def _mm_body(x_ref, w_ref, b_ref, o_ref, wbf_ref):
    @pl.when(pl.program_id(0) == 0)
    def _():
        wbf_ref[...] = w_ref[...].astype(jnp.bfloat16)
    xb = x_ref[...].astype(jnp.bfloat16)
    acc = lax.dot_general(xb, wbf_ref[...], (((1,), (1,)), ((), ())),
                          preferred_element_type=jnp.float32)
    o_ref[...] = acc + b_ref[...]


def _matmul(x, w, bias2):
    batch = x.shape[0]
    return pl.pallas_call(
        _mm_body,
        grid=(batch // BM,),
        in_specs=[
            pl.BlockSpec((BM, N_IN), lambda i: (i, 0)),
            pl.BlockSpec((N_OUT, N_IN), lambda i: (0, 0)),
            pl.BlockSpec((1, N_OUT), lambda i: (0, 0)),
        ],
        out_specs=pl.BlockSpec((BM, N_OUT), lambda i: (i, 0)),
        out_shape=jax.ShapeDtypeStruct((batch, N_OUT), jnp.float32),
        scratch_shapes=[pltpu.VMEM((N_OUT, N_IN), jnp.bfloat16)],
    )(x, w, bias2)


def kernel(in_values, values, indices, bias):
    ind = indices.astype(jnp.int32)
    nnz = values.shape[0]
    q = jnp.arange(0, N_OUT + 1, HALF_ROWS, dtype=jnp.int32)
    r_sub = indices[0, ::STRIDE].astype(jnp.int32)
    jc = jnp.sum(r_sub[None, :] < q[:, None], axis=1).astype(jnp.int32)
    coarse = jnp.pad(jc, (0, 128 - jc.shape[0]))
    trow = ind[0, nnz - 128:]
    tcol = ind[1, nnz - 128:]
    tval = values[nnz - 128:]
    w = _densify(coarse, ind, values, trow, tcol, tval)
    return _matmul(in_values, w, bias.reshape(1, N_OUT))


# speedup vs baseline: 1.3007x; 1.0043x over previous
import jax
import jax.numpy as jnp
from jax import lax
from jax.experimental import pallas as pl
from jax.experimental.pallas import tpu as pltpu
from jax.experimental.pallas import tpu_sc as plsc

N_IN = 2048
N_OUT = 2048
HALF_ROWS = 32
HALF_W = HALF_ROWS * N_IN
SUB = 8192
STRIDE = SUB


def _sc_densify_body(nnz, coarse_hbm, ind_hbm, vals_hbm,
                     trow_hbm, tcol_hbm, tval_hbm, z_hbm, w_hbm,
                     bnd_v, rcA, vA, rcB, vB,
                     tr_v, tc_v, tv_v, wbuf, semA, semB, semZ):
    c = lax.axis_index("c")
    s = lax.axis_index("s")
    t2 = (s * 2 + c) * 2
    pltpu.async_copy(z_hbm, wbuf, semZ)
    pltpu.sync_copy(coarse_hbm, bnd_v)
    pltpu.sync_copy(trow_hbm, tr_v)
    pltpu.sync_copy(tcol_hbm, tc_v)
    pltpu.sync_copy(tval_hbm, tv_v)
    zero16 = jnp.zeros((16,), jnp.float32)
    lanes0 = lax.iota(jnp.int32, 16)
    last128 = (nnz - SUB) & ~127
    wsmax = (nnz - STRIDE) & ~127
    TAIL = 128

    jcv = bnd_v[pl.ds(t2, 16)]
    ws_list = []
    for k in range(3):
        ws = jnp.minimum(jnp.maximum((jcv[k] - 1) * STRIDE, 0), wsmax)
        ws_list.append(pl.multiple_of(ws, 128))

    def count(buf, k):
        q = (t2 + k) * HALF_ROWS
        def cbody(i, acc):
            v = buf[0, pl.ds(i * 16, 16)]
            return acc + plsc.all_reduce_population_count(v < q)
        acc = lax.fori_loop(0, STRIDE // 16, cbody,
                            jnp.zeros((16,), jnp.int32), unroll=8)
        for i in range(TAIL // 16):
            tpos = lanes0 + (nnz - TAIL) + i * 16
            acc = acc + plsc.all_reduce_population_count(
                (tpos >= ws_list[k] + STRIDE)
                & (tr_v[pl.ds(i * 16, 16)] < q))
        return ws_list[k] + acc[0]

    h0 = pltpu.async_copy(ind_hbm.at[:, pl.ds(ws_list[0], STRIDE)], rcA, semA)
    h1 = pltpu.async_copy(ind_hbm.at[:, pl.ds(ws_list[1], STRIDE)], rcB, semB)
    h0.wait()
    b_0 = count(rcA, 0)
    h2 = pltpu.async_copy(ind_hbm.at[:, pl.ds(ws_list[2], STRIDE)], rcA, semA)
    h1.wait()
    b_1 = count(rcB, 1)
    h2.wait()
    b_2 = count(rcA, 2)
    bnds = [b_0, b_1, b_2]

    def starts(j, s0):
        rstart = jnp.minimum(s0 + j * SUB, last128)
        return pl.multiple_of(rstart, 128)

    def issue(rstart, rc_b, v_b, sem):
        pltpu.async_copy(ind_hbm.at[:, pl.ds(rstart, SUB)], rc_b, sem)
        pltpu.async_copy(vals_hbm.at[pl.ds(rstart, SUB)], v_b, sem)

    def drain(rc_b, v_b, sem):
        pltpu.make_async_copy(ind_hbm.at[:, pl.ds(0, SUB)], rc_b, sem).wait()
        pltpu.make_async_copy(vals_hbm.at[pl.ds(0, SUB)], v_b, sem).wait()

    for h in range(2):
        blk = t2 + h
        b0, b1 = bnds[h], bnds[h + 1]
        row0 = blk * HALF_ROWS
        s0 = b0 & ~127

        issue(starts(0, s0), rcA, vA, semA)
        pltpu.make_async_copy(z_hbm, wbuf, semZ).wait()

        def scatter(j, rc_b, v_b):
            rstart = starts(j, s0)
            lo = b0 - rstart
            hi = b1 - rstart

            G = 8

            def scat_masked(i, _):
                b = i * (G * 16)
                rs = [rc_b[0, pl.ds(b + g * 16, 16)] for g in range(G)]
                cs = [rc_b[1, pl.ds(b + g * 16, 16)] for g in range(G)]
                vs = [v_b[pl.ds(b + g * 16, 16)] for g in range(G)]
                for g in range(G):
                    lane = lanes0 + (b + g * 16)
                    m = (lane >= lo) & (lane < hi)
                    plsc.store_scatter(wbuf, [rs[g] - row0, cs[g]], vs[g],
                                       mask=m)
                return 0

            def scat_full(i, _):
                b = i * (G * 16)
                rs = [rc_b[0, pl.ds(b + g * 16, 16)] for g in range(G)]
                cs = [rc_b[1, pl.ds(b + g * 16, 16)] for g in range(G)]
                vs = [v_b[pl.ds(b + g * 16, 16)] for g in range(G)]
                for g in range(G):
                    plsc.store_scatter(wbuf, [rs[g] - row0, cs[g]], vs[g])
                return 0

            lax.cond(
                (lo <= 0) & (hi >= SUB),
                lambda: lax.fori_loop(0, SUB // (G * 16), scat_full, 0,
                                      unroll=2),
                lambda: lax.fori_loop(0, SUB // (G * 16), scat_masked, 0,
                                      unroll=2))

        nsub = (b1 - s0 + SUB - 1) // SUB

        def pair(jj, _):
            j0 = 2 * jj
            issue(starts(j0 + 1, s0), rcB, vB, semB)
            drain(rcA, vA, semA)
            scatter(j0, rcA, vA)
            issue(starts(j0 + 2, s0), rcA, vA, semA)
            drain(rcB, vB, semB)
            scatter(j0 + 1, rcB, vB)
            return 0
        pairs = nsub // 2
        lax.fori_loop(0, pairs, pair, 0)
        drain(rcA, vA, semA)
        lax.cond(nsub % 2 == 1,
                 lambda: scatter(2 * pairs, rcA, vA), lambda: None)

        for i in range(TAIL // 16):
            trow = tr_v[pl.ds(i * 16, 16)]
            tmask = (trow >= row0) & (trow < row0 + HALF_ROWS)
            plsc.store_scatter(
                wbuf, [trow - row0, tc_v[pl.ds(i * 16, 16)]],
                tv_v[pl.ds(i * 16, 16)], mask=tmask)

        pltpu.sync_copy(wbuf, w_hbm.at[pl.ds(row0, HALF_ROWS), :])
        if h == 0:
            pltpu.async_copy(z_hbm, wbuf, semZ)


def _densify(coarse, ind, vals, trow, tcol, tval):
    mesh = plsc.VectorSubcoreMesh(core_axis_name="c", subcore_axis_name="s")
    nnz = vals.shape[0]
    body = lambda *refs: _sc_densify_body(nnz, *refs)
    return pl.kernel(
        body,
        out_type=jax.ShapeDtypeStruct((N_OUT, N_IN), jnp.float32),
        mesh=mesh,
        scratch_types=[
            pltpu.VMEM((128,), jnp.int32),
            pltpu.VMEM((2, SUB), jnp.int32),
            pltpu.VMEM((SUB,), jnp.float32),
            pltpu.VMEM((2, SUB), jnp.int32),
            pltpu.VMEM((SUB,), jnp.float32),
            pltpu.VMEM((128,), jnp.int32),
            pltpu.VMEM((128,), jnp.int32),
            pltpu.VMEM((128,), jnp.float32),
            pltpu.VMEM((HALF_ROWS, N_IN), jnp.float32),
            pltpu.SemaphoreType.DMA,
            pltpu.SemaphoreType.DMA,
            pltpu.SemaphoreType.DMA,
        ],
        compiler_params=pltpu.CompilerParams(needs_layout_passes=False),
    )(coarse, ind, vals, trow, tcol, tval,
      jnp.zeros((HALF_ROWS, N_IN), jnp.float32))


BM = 512


def _mm_body(x_ref, w_ref, b_ref, o_ref, wbf_ref):
    @pl.when(pl.program_id(0) == 0)
    def _():
        wbf_ref[...] = w_ref[...].astype(jnp.bfloat16)
    xb = x_ref[...].astype(jnp.bfloat16)
    acc = lax.dot_general(xb, wbf_ref[...], (((1,), (1,)), ((), ())),
                          preferred_element_type=jnp.float32)
    o_ref[...] = acc + b_ref[...]


def _matmul(x, w, bias2):
    batch = x.shape[0]
    return pl.pallas_call(
        _mm_body,
        grid=(batch // BM,),
        in_specs=[
            pl.BlockSpec((BM, N_IN), lambda i: (i, 0)),
            pl.BlockSpec((N_OUT, N_IN), lambda i: (0, 0)),
            pl.BlockSpec((1, N_OUT), lambda i: (0, 0)),
        ],
        out_specs=pl.BlockSpec((BM, N_OUT), lambda i: (i, 0)),
        out_shape=jax.ShapeDtypeStruct((batch, N_OUT), jnp.float32),
        scratch_shapes=[pltpu.VMEM((N_OUT, N_IN), jnp.bfloat16)],
    )(x, w, bias2)


def kernel(in_values, values, indices, bias):
    ind = indices.astype(jnp.int32)
    nnz = values.shape[0]
    q = jnp.arange(0, N_OUT + 1, HALF_ROWS, dtype=jnp.int32)
    r_sub = indices[0, ::SUB].astype(jnp.int32)
    jc = jnp.sum(r_sub[None, :] < q[:, None], axis=1).astype(jnp.int32)
    coarse = jnp.pad(jc, (0, 128 - jc.shape[0]))
    trow = ind[0, nnz - 128:]
    tcol = ind[1, nnz - 128:]
    tval = values[nnz - 128:]
    w = _densify(coarse, ind, values, trow, tcol, tval)
    return _matmul(in_values, w, bias.reshape(1, N_OUT))
